# Initial kernel scaffold; baseline (speedup 1.0000x reference)
import jax
import jax.numpy as jnp
from jax.experimental import pallas as pl


def kernel(h, pos, edge_index, batch, W1, Wpos, W2, Wm1, bm1, Wm2, bm2, Wq, Wk):
    raise NotImplementedError("write your pallas kernel here")


# speedup vs baseline: 1.2529x; 1.2529x over previous
import jax
import jax.numpy as jnp
from jax import lax
from jax.experimental import pallas as pl
from jax.experimental.pallas import tpu as pltpu
from jax.experimental.pallas import tpu_sc as plsc

N = 10000
E = 320000
H_DIM = 128
S_DIM = 64
POS_DIM = 3
K_MASTER = 500
LAMBDA_ATTN = 0.05

_NC = 2
_NS = 16
_NW = _NC * _NS
_EPW = E // _NW
_CHUNK = 80
_L = 10160
_NCHUNK = _L // _CHUNK
_NPAD = 10240
_STRIPE = _NPAD // _NS
_SENT = _NPAD - 1
_NB = H_DIM // 16


def _rne_bf16(x):
    u = lax.bitcast_convert_type(x, jnp.int32)
    low = lax.shift_right_logical(u, 16) & 1
    u = (u + 32767 + low) & jnp.int32(-65536)
    return lax.bitcast_convert_type(u, jnp.float32)


def _pre_body(h_ref, w1_ref, hw_ref):
    hw_ref[...] = jnp.dot(h_ref[...], w1_ref[...],
                          preferred_element_type=jnp.float32)


def _precompute(h, w1):
    R = 1000
    return pl.pallas_call(
        _pre_body,
        grid=(N // R,),
        in_specs=[
            pl.BlockSpec((R, H_DIM), lambda i: (i, 0)),
            pl.BlockSpec((H_DIM, H_DIM), lambda i: (0, 0)),
        ],
        out_specs=pl.BlockSpec((R, H_DIM), lambda i: (i, 0)),
        out_shape=jax.ShapeDtypeStruct((N, H_DIM), jnp.float32),
    )(h, w1)


def _edge_body(hw_hbm, pos_hbm, wpr_hbm, src_hbm, dst_hbm, out_hbm,
               sidx, didx, hw_buf, ps_buf, pd_buf, wp_buf, zbuf, rowbuf,
               dsm, acc, sem, sem2, sem3):
    c = lax.axis_index("c")
    s = lax.axis_index("s")
    wid = s * _NC + c

    pltpu.sync_copy(wpr_hbm, wp_buf)

    zv = jnp.zeros((16,), jnp.float32)

    def zb(i, _):
        zbuf[pl.ds(i * 16, 16)] = zv
        return 0

    lax.fori_loop(0, zbuf.shape[0] // 16, zb, 0)
    row0 = pl.multiple_of(s * _STRIPE * H_DIM, 8)
    for j in range(_STRIPE * H_DIM // zbuf.shape[0]):
        pltpu.sync_copy(zbuf,
                        acc.at[pl.ds(pl.multiple_of(row0 + j * zbuf.shape[0],
                                                    8), zbuf.shape[0])])
    plsc.subcore_barrier()

    wv = [[wp_buf[p, pl.ds(r * 16, 16)] for p in range(POS_DIM)]
          for r in range(_NB)]

    def flush(cur, racc):
        @pl.when(cur >= 0)
        def _():
            for r in range(_NB):
                rowbuf[pl.ds(r * 16, 16)] = racc[r]
            pltpu.sync_copy(rowbuf,
                            acc.at[pl.ds(pl.multiple_of(cur * H_DIM, 8),
                                         H_DIM)])

    def chunk_body(i, carry):
        ebase = pl.multiple_of(wid * _L + i * _CHUNK, 8)
        pltpu.sync_copy(src_hbm.at[pl.ds(ebase, _CHUNK)], sidx)
        pltpu.sync_copy(dst_hbm.at[pl.ds(ebase, _CHUNK)], didx)
        da = pltpu.async_copy(hw_hbm.at[sidx], hw_buf, sem)
        db = pltpu.async_copy(pos_hbm.at[sidx], ps_buf, sem2)
        dc = pltpu.async_copy(pos_hbm.at[didx], pd_buf, sem3)
        da.wait()
        db.wait()
        dc.wait()

        for g in range(_CHUNK // 16):
            dv = didx[pl.ds(g * 16, 16)]
            for l in range(16):
                dsm[g * 16 + l] = dv[l]

        def ebody(e, ecarry):
            cur = ecarry[0]
            racc = list(ecarry[1:])
            d = dsm[e]
            rel = pd_buf[e, pl.ds(0, 16)] - ps_buf[e, pl.ds(0, 16)]
            relr = _rne_bf16(rel)
            rb = [relr[p] for p in range(POS_DIM)]
            msg = []
            for r in range(_NB):
                pw = (rb[0] * wv[r][0] + rb[1] * wv[r][1]) + rb[2] * wv[r][2]
                msg.append(jnp.maximum(hw_buf[e, pl.ds(r * 16, 16)] + pw, 0.0))
            is_new = d != cur

            @pl.when(is_new)
            def _():
                flush(cur, racc)

            racc = [jnp.where(is_new, msg[r], racc[r] + msg[r])
                    for r in range(_NB)]
            return (d, *racc)

        return lax.fori_loop(0, _CHUNK, ebody, carry)

    z8 = [jnp.zeros((16,), jnp.float32)] * _NB
    carry = lax.fori_loop(0, _NCHUNK, chunk_body, (jnp.int32(-1), *z8))
    flush(carry[0], list(carry[1:]))
    plsc.subcore_barrier()

    pltpu.sync_copy(acc.at[pl.ds(row0, _STRIPE * H_DIM)],
                    out_hbm.at[c, pl.ds(row0, _STRIPE * H_DIM)])


def _edge_agg(hw, pos128, wpr, src_w, dst_w):
    mesh = plsc.VectorSubcoreMesh(core_axis_name="c", subcore_axis_name="s")
    return pl.kernel(
        _edge_body,
        out_type=jax.ShapeDtypeStruct((_NC, _NPAD * H_DIM), jnp.float32),
        mesh=mesh,
        scratch_types=[
            pltpu.VMEM((_CHUNK,), jnp.int32),
            pltpu.VMEM((_CHUNK,), jnp.int32),
            pltpu.VMEM((_CHUNK, H_DIM), jnp.float32),
            pltpu.VMEM((_CHUNK, H_DIM), jnp.float32),
            pltpu.VMEM((_CHUNK, H_DIM), jnp.float32),
            pltpu.VMEM((POS_DIM, H_DIM), jnp.float32),
            pltpu.VMEM((2560,), jnp.float32),
            pltpu.VMEM((H_DIM,), jnp.float32),
            pltpu.SMEM((_CHUNK,), jnp.int32),
            pltpu.VMEM_SHARED((_NPAD * H_DIM,), jnp.float32),
            pltpu.SemaphoreType.DMA,
            pltpu.SemaphoreType.DMA,
            pltpu.SemaphoreType.DMA,
        ],
    )(hw, pos128, wpr, src_w, dst_w)


def _comb_body(agg0_ref, agg1_ref, h_ref, w2_ref, wm1_ref, bm1_ref,
               wm2_ref, bm2_ref, hl_ref, sc_ref):
    f32 = jnp.float32
    agg = agg0_ref[...] + agg1_ref[...]
    hu = jnp.dot(agg, w2_ref[...], preferred_element_type=f32)
    hl = h_ref[...] + hu
    hl_ref[...] = hl
    hs = hl[:, :S_DIM]
    hid = jnp.tanh(jnp.dot(hs, wm1_ref[...], preferred_element_type=f32)
                   + bm1_ref[...])
    z = jnp.dot(hid, wm2_ref[...], preferred_element_type=f32) + bm2_ref[...]
    sc_ref[...] = jax.nn.sigmoid(z)


def _combine(agg0, agg1, h, w2, wm1, bm1_row, wm2p, bm2_row):
    R = 1000
    return pl.pallas_call(
        _comb_body,
        grid=(N // R,),
        in_specs=[
            pl.BlockSpec((R, H_DIM), lambda i: (i, 0)),
            pl.BlockSpec((R, H_DIM), lambda i: (i, 0)),
            pl.BlockSpec((R, H_DIM), lambda i: (i, 0)),
            pl.BlockSpec((H_DIM, H_DIM), lambda i: (0, 0)),
            pl.BlockSpec((S_DIM, S_DIM), lambda i: (0, 0)),
            pl.BlockSpec((1, S_DIM), lambda i: (0, 0)),
            pl.BlockSpec((S_DIM, H_DIM), lambda i: (0, 0)),
            pl.BlockSpec((1, H_DIM), lambda i: (0, 0)),
        ],
        out_specs=[
            pl.BlockSpec((R, H_DIM), lambda i: (i, 0)),
            pl.BlockSpec((R, H_DIM), lambda i: (i, 0)),
        ],
        out_shape=[
            jax.ShapeDtypeStruct((N, H_DIM), jnp.float32),
            jax.ShapeDtypeStruct((N, H_DIM), jnp.float32),
        ],
    )(agg0, agg1, h, w2, wm1, bm1_row, wm2p, bm2_row)


def kernel(h, pos, edge_index, batch, W1, Wpos, W2, Wm1, bm1, Wm2, bm2, Wq, Wk):
    f32 = jnp.float32
    src = edge_index[0]
    dst = edge_index[1]

    hw = _precompute(h, W1)
    pos128 = jnp.zeros((N, H_DIM), f32).at[:, :POS_DIM].set(pos)
    wpr = _rne_bf16(Wpos)

    perm = jnp.argsort(dst, stable=True)
    src_s = src[perm]
    dst_s = dst[perm]
    node_starts = jnp.searchsorted(dst_s, jnp.arange(N, dtype=jnp.int32))
    tgt = jnp.arange(_NW, dtype=jnp.int32) * _EPW
    nidx = jnp.searchsorted(node_starts, tgt)
    starts = jnp.where(nidx < N, node_starts[jnp.minimum(nidx, N - 1)], E)
    ends = jnp.concatenate([starts[1:], jnp.asarray([E], starts.dtype)])
    src_p = jnp.concatenate([src_s, jnp.zeros((_L,), jnp.int32)])
    dst_p = jnp.concatenate([dst_s, jnp.full((_L,), _SENT, jnp.int32)])

    def slc(st, en):
        sw = lax.dynamic_slice(src_p, (st,), (_L,))
        dw = lax.dynamic_slice(dst_p, (st,), (_L,))
        valid = jnp.arange(_L) < (en - st)
        return jnp.where(valid, sw, 0), jnp.where(valid, dw, _SENT)

    src_w, dst_w = jax.vmap(slc)(starts, ends)

    agg2 = _edge_agg(hw, pos128, wpr, src_w.reshape(-1), dst_w.reshape(-1))
    agg2 = agg2.reshape(_NC, _NPAD, H_DIM)

    bm1_row = bm1.reshape(1, S_DIM)
    wm2p = jnp.zeros((S_DIM, H_DIM), f32).at[:, :1].set(Wm2)
    bm2_row = jnp.broadcast_to(bm2.reshape(1, 1), (1, H_DIM))
    h_local, sc_full = _combine(agg2[0, :N], agg2[1, :N], h, W2, Wm1, bm1_row,
                                wm2p, bm2_row)
    scores = sc_full[:, 0]

    topv, topi = lax.top_k(scores, K_MASTER)
    m = jnp.zeros((N,), f32).at[topi].set(topv)
    node2master = jnp.full((N,), K_MASTER, dtype=jnp.int32).at[topi].set(
        jnp.arange(K_MASTER, dtype=jnp.int32))
    sm = node2master[src]
    dm = node2master[dst]
    valid = ((sm < K_MASTER) & (dm < K_MASTER)).astype(f32)
    adj_counts = jnp.zeros((K_MASTER + 1, K_MASTER + 1), f32).at[sm, dm].add(
        valid)[:K_MASTER, :K_MASTER]
    adj_induced = (adj_counts > 0).astype(f32)
    h_master = h_local[topi]
    pos_master = pos[topi]

    hms = h_master[:, :S_DIM]
    q = hms @ Wq
    k_ = hms @ Wk
    attn = jax.nn.softmax(q @ k_.T / jnp.sqrt(jnp.asarray(S_DIM, f32)),
                          axis=-1)
    eye = jnp.eye(K_MASTER, dtype=bool)
    new_mask = (adj_induced == 0) & (~eye) & (attn > LAMBDA_ATTN)
    A_virtual = attn * new_mask.astype(f32)
    A_comb = ((adj_induced + (A_virtual > 0).astype(f32)) > 0).astype(f32)

    hw_m = h_master @ W1
    rel = pos_master[None, :, :] - pos_master[:, None, :]
    posw = jnp.einsum('jip,ph->jih', rel, Wpos)
    msg = jax.nn.relu(hw_m[:, None, :] + posw)
    agg_m = jnp.einsum('ji,jih->ih', A_comb, msg)
    h_master_update = agg_m @ W2
    h_hier = h_master + h_master_update

    h_exp = jnp.zeros_like(h_local).at[topi].set(h_hier)
    m_e = m[:, None]
    h_final = (1.0 - m_e) * h_local + m_e * h_exp
    return h_final, pos, A_virtual, m
